# baseline (speedup 1.0000x reference)
import functools

import jax
import jax.numpy as jnp
from jax import lax
from jax.experimental import pallas as pl
from jax.experimental.pallas import tpu as pltpu
from jax.experimental.pallas import tpu_sc as plsc

NC = 2
NS = 16
L = 16

CH = 80


def _pre_body(x_ref, w1_ref, w2_ref, b_ref, p_ref, q_ref):
    x = x_ref[...]
    p_ref[...] = jnp.dot(
        x, w1_ref[...], preferred_element_type=jnp.float32
    ).astype(jnp.bfloat16)
    q_ref[...] = (
        jnp.dot(x, w2_ref[...], preferred_element_type=jnp.float32)
        + b_ref[...]
    ).astype(jnp.bfloat16)


def _pack_i32(a):
    n, d = a.shape
    return jax.lax.bitcast_convert_type(a.reshape(n, d // 2, 2), jnp.int32)


def _upd_body(agg_ref, x_ref, w1_ref, w2_ref, b_ref, o_ref):
    n = x_ref.shape[0]
    a = agg_ref[0, :n] + agg_ref[1, :n]
    o_ref[...] = jnp.maximum(
        jnp.dot(a, w1_ref[...], preferred_element_type=jnp.float32)
        + jnp.dot(x_ref[...], w2_ref[...], preferred_element_type=jnp.float32)
        + b_ref[...],
        0.0,
    )


def _make_sc_edge(N, D, E):
    assert E % (NC * NS) == 0
    ew = E // (NC * NS)
    assert ew % CH == 0
    n_chunks = ew // CH
    n_pad = -(-N // (NS * CH)) * (NS * CH)
    rows_w = n_pad // NS
    st = CH
    n_st = rows_w // st
    assert st % 8 == 0 and rows_w % st == 0

    mesh = plsc.VectorSubcoreMesh(
        core_axis_name="c", subcore_axis_name="s",
        num_cores=NC, num_subcores=NS,
    )

    SBC = 25
    assert n_chunks % SBC == 0 and SBC % 2 == 1 and SBC >= 3
    n_sup = n_chunks // SBC
    kk = (SBC - 3) // 2

    @functools.partial(
        pl.kernel,
        out_type=jax.ShapeDtypeStruct((NC, n_pad, D), jnp.float32),
        mesh=mesh,
        compiler_params=pltpu.CompilerParams(use_tc_tiling_on_sc=False),
        scratch_types=[
            pltpu.VMEM((SBC * CH,), jnp.int32),
            pltpu.VMEM((SBC * CH,), jnp.int32),
            pltpu.VMEM((CH,), jnp.int32),
            pltpu.VMEM((CH,), jnp.int32),
            pltpu.VMEM((CH, D // 2), jnp.int32),
            pltpu.VMEM((CH, D // 2), jnp.int32),
            pltpu.VMEM((CH, D // 2), jnp.int32),
            pltpu.VMEM((CH, D // 2), jnp.int32),
            pltpu.VMEM((CH, D), jnp.float32),
            pltpu.VMEM((CH, D), jnp.float32),
            pltpu.VMEM_SHARED((n_pad, D), jnp.float32),
            pltpu.SemaphoreType.DMA,
            pltpu.SemaphoreType.DMA,
        ],
    )
    def sc_edge(p_hbm, q_hbm, src_hbm, dst_hbm, out_hbm,
                sidx_sc, didx_sc, didx0, didx1, prow0, prow1, qrow0, qrow1,
                msg0, msg1, agg_sh, semg, sems):
        cid = lax.axis_index("c")
        sid = lax.axis_index("s")
        dxb = [didx0, didx1]
        pr = [prow0, prow1]
        qr = [qrow0, qrow1]
        ms = [msg0, msg1]

        def zrow(r, _):
            for k in range(D // L):
                msg0[r, pl.ds(k * L, L)] = jnp.zeros((L,), jnp.float32)
            return 0
        lax.fori_loop(0, st, zrow, 0)
        for j in range(n_st):
            pltpu.sync_copy(msg0, agg_sh.at[pl.ds(sid * rows_w + j * st, st)])
        plsc.subcore_barrier()

        base_w = (cid * NS + sid) * ew

        def load_super(s):
            base = base_w + s * (SBC * CH)
            pltpu.sync_copy(src_hbm.at[pl.ds(base, SBC * CH)], sidx_sc)
            pltpu.sync_copy(dst_hbm.at[pl.ds(base, SBC * CH)], didx_sc)

        def fire_gather(c, b):
            pltpu.async_copy(
                p_hbm.at[sidx_sc.at[pl.ds(c * CH, CH)]], pr[b], semg)
            pltpu.async_copy(
                q_hbm.at[didx_sc.at[pl.ds(c * CH, CH)]], qr[b], semg)

        def wait_gather(b):
            pltpu.make_async_copy(
                p_hbm.at[sidx_sc.at[pl.ds(0, CH)]], pr[b], semg).wait()
            pltpu.make_async_copy(
                q_hbm.at[didx_sc.at[pl.ds(0, CH)]], qr[b], semg).wait()

        def fire_scatter(b):
            pltpu.async_copy(ms[b], agg_sh.at[dxb[b]], sems, add=True)

        def wait_scatter(b):
            pltpu.make_async_copy(ms[b], agg_sh.at[dxb[b]], sems).wait()

        def stage_didx(c, b):
            for k in range(CH // L):
                dxb[b][pl.ds(k * L, L)] = didx_sc[pl.ds(c * CH + k * L, L)]

        def compute(b):
            hmask = jnp.full((L,), -65536, jnp.int32)
            @plsc.parallel_loop(0, CH, unroll=2)
            def _(r):
                for g in range(D // (2 * L)):
                    s2 = pl.ds(g * L, L)
                    pw = pr[b][r, s2]
                    qw = qr[b][r, s2]
                    pa = lax.bitcast_convert_type(pw << 16, jnp.float32)
                    pb_ = lax.bitcast_convert_type(pw & hmask, jnp.float32)
                    qa = lax.bitcast_convert_type(qw << 16, jnp.float32)
                    qb_ = lax.bitcast_convert_type(qw & hmask, jnp.float32)
                    lo = pl.ds(g * 2 * L, L)
                    hi = pl.ds(g * 2 * L + L, L)
                    ms[b][r, lo] = jnp.maximum(pa + qa, 0.0)
                    ms[b][r, hi] = jnp.maximum(pb_ + qb_, 0.0)

        def full_step(c, b, prefetch, pending_scatter):
            if prefetch:
                if pending_scatter:
                    wait_scatter(1 - b)
                fire_gather(c + 1, 1 - b)
            wait_gather(b)
            stage_didx(c, b)
            compute(b)
            fire_scatter(b)

        def super_body(s, first):
            load_super(s)
            if not first:
                wait_scatter(0)
            fire_gather(0, 0)
            full_step(0, 0, prefetch=True, pending_scatter=not first)

            def body(j, _):
                c = 2 * j + 1
                full_step(c, 1, prefetch=True, pending_scatter=True)
                full_step(c + 1, 0, prefetch=True, pending_scatter=True)
                return 0
            lax.fori_loop(0, kk, body, 0)

            full_step(SBC - 2, 1, prefetch=True, pending_scatter=True)
            full_step(SBC - 1, 0, prefetch=False, pending_scatter=False)

        super_body(0, first=True)

        def sbody(s, _):
            super_body(s, first=False)
            return 0
        lax.fori_loop(1, n_sup, sbody, 0)
        wait_scatter(1)
        wait_scatter(0)

        plsc.subcore_barrier()
        for j in range(n_st):
            off = sid * rows_w + j * st
            pltpu.sync_copy(agg_sh.at[pl.ds(off, st)], msg0)
            pltpu.sync_copy(msg0, out_hbm.at[cid, pl.ds(off, st)])

    return sc_edge


def kernel(x, edge_index, W_msg, b_msg, W_upd, b_upd):
    N, D = x.shape
    E = edge_index.shape[1]

    P, Q = pl.pallas_call(
        _pre_body,
        out_shape=[jax.ShapeDtypeStruct((N, D), jnp.bfloat16)] * 2,
    )(x, W_msg[:D], W_msg[D:], b_msg.reshape(1, D))

    agg = _make_sc_edge(N, D, E)(
        _pack_i32(P), _pack_i32(Q), edge_index[0], edge_index[1])

    g = jnp.arange(D).reshape(D // 32, 32) // 32 * 32
    u = jnp.arange(D).reshape(D // 32, 32) % 32
    sigma = (g + jnp.where(u < 16, 2 * u, 2 * (u - 16) + 1)).reshape(D)
    out = pl.pallas_call(
        _upd_body,
        out_shape=jax.ShapeDtypeStruct((N, D), jnp.float32),
    )(agg, x, W_upd[:D][sigma], W_upd[D:], b_upd.reshape(1, D))
    return out


# speedup vs baseline: 1.0394x; 1.0394x over previous
import functools

import jax
import jax.numpy as jnp
from jax import lax
from jax.experimental import pallas as pl
from jax.experimental.pallas import tpu as pltpu
from jax.experimental.pallas import tpu_sc as plsc

NC = 2
NS = 16
L = 16

CH = 80


def _pre_body(x_ref, w1_ref, w2_ref, b_ref, p_ref, q_ref):
    x = x_ref[...]
    p_ref[...] = jnp.dot(x, w1_ref[...], preferred_element_type=jnp.float32)
    q_ref[...] = (
        jnp.dot(x, w2_ref[...], preferred_element_type=jnp.float32)
        + b_ref[...]
    )


def _upd_body(agg_ref, x_ref, w1_ref, w2_ref, b_ref, o_ref):
    n = x_ref.shape[0]
    a = agg_ref[0, :n] + agg_ref[1, :n]
    o_ref[...] = jnp.maximum(
        jnp.dot(a, w1_ref[...], preferred_element_type=jnp.float32)
        + jnp.dot(x_ref[...], w2_ref[...], preferred_element_type=jnp.float32)
        + b_ref[...],
        0.0,
    )


def _make_sc_edge(N, D, E):
    assert E % (NC * NS) == 0
    ew = E // (NC * NS)
    assert ew % CH == 0
    n_chunks = ew // CH
    n_pad = -(-N // (NS * CH)) * (NS * CH)
    rows_w = n_pad // NS
    st = CH
    n_st = rows_w // st
    assert st % 8 == 0 and rows_w % st == 0

    mesh = plsc.VectorSubcoreMesh(
        core_axis_name="c", subcore_axis_name="s",
        num_cores=NC, num_subcores=NS,
    )

    SBC = 25
    assert n_chunks % SBC == 0 and SBC % 2 == 1 and SBC >= 3
    n_sup = n_chunks // SBC
    kk = (SBC - 3) // 2

    @functools.partial(
        pl.kernel,
        out_type=jax.ShapeDtypeStruct((NC, n_pad, D), jnp.float32),
        mesh=mesh,
        scratch_types=[
            pltpu.VMEM((SBC * CH,), jnp.int32),
            pltpu.VMEM((SBC * CH,), jnp.int32),
            pltpu.VMEM((CH,), jnp.int32),
            pltpu.VMEM((CH,), jnp.int32),
            pltpu.VMEM((CH, D), jnp.float32),
            pltpu.VMEM((CH, D), jnp.float32),
            pltpu.VMEM((CH, D), jnp.float32),
            pltpu.VMEM((CH, D), jnp.float32),
            pltpu.VMEM_SHARED((n_pad, D), jnp.float32),
            pltpu.SemaphoreType.DMA,
            pltpu.SemaphoreType.DMA,
        ],
    )
    def sc_edge(p_hbm, q_hbm, src_hbm, dst_hbm, out_hbm,
                sidx_sc, didx_sc, didx0, didx1, prow0, prow1, qrow0, qrow1,
                agg_sh, semg, sems):
        cid = lax.axis_index("c")
        sid = lax.axis_index("s")
        dxb = [didx0, didx1]
        pr = [prow0, prow1]
        qr = [qrow0, qrow1]

        def zrow(r, _):
            for k in range(D // L):
                prow0[r, pl.ds(k * L, L)] = jnp.zeros((L,), jnp.float32)
            return 0
        lax.fori_loop(0, st, zrow, 0)
        for j in range(n_st):
            pltpu.sync_copy(prow0, agg_sh.at[pl.ds(sid * rows_w + j * st, st)])
        plsc.subcore_barrier()

        base_w = (cid * NS + sid) * ew

        def load_super(s):
            base = base_w + s * (SBC * CH)
            pltpu.sync_copy(src_hbm.at[pl.ds(base, SBC * CH)], sidx_sc)
            pltpu.sync_copy(dst_hbm.at[pl.ds(base, SBC * CH)], didx_sc)

        def fire_gather(c, b):
            pltpu.async_copy(
                p_hbm.at[sidx_sc.at[pl.ds(c * CH, CH)]], pr[b], semg)
            pltpu.async_copy(
                q_hbm.at[didx_sc.at[pl.ds(c * CH, CH)]], qr[b], semg)

        def wait_gather(b):
            pltpu.make_async_copy(
                p_hbm.at[sidx_sc.at[pl.ds(0, CH)]], pr[b], semg).wait()
            pltpu.make_async_copy(
                q_hbm.at[didx_sc.at[pl.ds(0, CH)]], qr[b], semg).wait()

        def fire_scatter(b):
            pltpu.async_copy(pr[b], agg_sh.at[dxb[b]], sems, add=True)

        def wait_scatter(b):
            pltpu.make_async_copy(pr[b], agg_sh.at[dxb[b]], sems).wait()

        def stage_didx(c, b):
            for k in range(CH // L):
                dxb[b][pl.ds(k * L, L)] = didx_sc[pl.ds(c * CH + k * L, L)]

        def compute(b):
            @plsc.parallel_loop(0, CH, unroll=2)
            def _(r):
                for k in range(D // L):
                    s = pl.ds(k * L, L)
                    pr[b][r, s] = jnp.maximum(pr[b][r, s] + qr[b][r, s], 0.0)

        def full_step(c, b, prefetch, pending_scatter):
            if prefetch:
                if pending_scatter:
                    wait_scatter(1 - b)
                fire_gather(c + 1, 1 - b)
            wait_gather(b)
            stage_didx(c, b)
            compute(b)
            fire_scatter(b)

        def super_body(s, first):
            load_super(s)
            if not first:
                wait_scatter(0)
            fire_gather(0, 0)
            full_step(0, 0, prefetch=True, pending_scatter=not first)

            def body(j, _):
                c = 2 * j + 1
                full_step(c, 1, prefetch=True, pending_scatter=True)
                full_step(c + 1, 0, prefetch=True, pending_scatter=True)
                return 0
            lax.fori_loop(0, kk, body, 0)

            full_step(SBC - 2, 1, prefetch=True, pending_scatter=True)
            full_step(SBC - 1, 0, prefetch=False, pending_scatter=False)

        super_body(0, first=True)

        def sbody(s, _):
            super_body(s, first=False)
            return 0
        lax.fori_loop(1, n_sup, sbody, 0)
        wait_scatter(1)
        wait_scatter(0)

        plsc.subcore_barrier()
        for j in range(n_st):
            off = sid * rows_w + j * st
            buf = pr[j % 2]
            if j >= 2:
                pltpu.make_async_copy(
                    buf, out_hbm.at[cid, pl.ds(0, st)], semg).wait()
            pltpu.sync_copy(agg_sh.at[pl.ds(off, st)], buf)
            pltpu.async_copy(buf, out_hbm.at[cid, pl.ds(off, st)], semg)
        pltpu.make_async_copy(
            prow0, out_hbm.at[cid, pl.ds(0, st)], semg).wait()
        pltpu.make_async_copy(
            prow1, out_hbm.at[cid, pl.ds(0, st)], semg).wait()

    return sc_edge


def kernel(x, edge_index, W_msg, b_msg, W_upd, b_upd):
    N, D = x.shape
    E = edge_index.shape[1]

    P, Q = pl.pallas_call(
        _pre_body,
        out_shape=[jax.ShapeDtypeStruct((N, D), jnp.float32)] * 2,
    )(x, W_msg[:D], W_msg[D:], b_msg.reshape(1, D))

    agg = _make_sc_edge(N, D, E)(P, Q, edge_index[0], edge_index[1])

    out = pl.pallas_call(
        _upd_body,
        out_shape=jax.ShapeDtypeStruct((N, D), jnp.float32),
    )(agg, x, W_upd[:D], W_upd[D:], b_upd.reshape(1, D))
    return out


# speedup vs baseline: 1.0835x; 1.0424x over previous
import functools

import jax
import jax.numpy as jnp
from jax import lax
from jax.experimental import pallas as pl
from jax.experimental.pallas import tpu as pltpu
from jax.experimental.pallas import tpu_sc as plsc

NC = 2
NS = 16
L = 16

CH = 80


def _pre_body(x_ref, w1_ref, w2_ref, b_ref, p_ref, q_ref):
    x = x_ref[...]
    p_ref[...] = jnp.dot(x, w1_ref[...], preferred_element_type=jnp.float32)
    q_ref[...] = (
        jnp.dot(x, w2_ref[...], preferred_element_type=jnp.float32)
        + b_ref[...]
    )


def _upd_body(agg_ref, x_ref, w1_ref, w2_ref, b_ref, o_ref):
    n = x_ref.shape[0]
    a = agg_ref[0, :n] + agg_ref[1, :n]
    o_ref[...] = jnp.maximum(
        jnp.dot(a, w1_ref[...], preferred_element_type=jnp.float32)
        + jnp.dot(x_ref[...], w2_ref[...], preferred_element_type=jnp.float32)
        + b_ref[...],
        0.0,
    )


def _make_sc_edge(N, D, E):
    assert E % (NC * NS) == 0
    ew = E // (NC * NS)
    assert ew % CH == 0
    n_chunks = ew // CH
    n_pad = -(-N // (NS * CH)) * (NS * CH)
    rows_w = n_pad // NS
    st = CH
    n_st = rows_w // st
    assert st % 8 == 0 and rows_w % st == 0

    mesh = plsc.VectorSubcoreMesh(
        core_axis_name="c", subcore_axis_name="s",
        num_cores=NC, num_subcores=NS,
    )

    SBC = 25
    assert n_chunks % SBC == 0 and SBC % 2 == 1 and SBC >= 3
    n_sup = n_chunks // SBC
    kk = (SBC - 3) // 2

    @functools.partial(
        pl.kernel,
        out_type=jax.ShapeDtypeStruct((NC, n_pad, D), jnp.float32),
        mesh=mesh,
        scratch_types=[
            pltpu.VMEM((SBC * CH,), jnp.int32),
            pltpu.VMEM((SBC * CH,), jnp.int32),
            pltpu.VMEM((CH,), jnp.int32),
            pltpu.VMEM((CH,), jnp.int32),
            pltpu.VMEM((CH, D), jnp.float32),
            pltpu.VMEM((CH, D), jnp.float32),
            pltpu.VMEM((CH, D), jnp.float32),
            pltpu.VMEM((CH, D), jnp.float32),
            pltpu.VMEM_SHARED((n_pad, D), jnp.float32),
            pltpu.SemaphoreType.DMA,
            pltpu.SemaphoreType.DMA,
        ],
    )
    def sc_edge(p_hbm, q_hbm, ei_hbm, out_hbm,
                sidx_sc, didx_sc, didx0, didx1, prow0, prow1, qrow0, qrow1,
                agg_sh, semg, sems):
        cid = lax.axis_index("c")
        sid = lax.axis_index("s")
        dxb = [didx0, didx1]
        pr = [prow0, prow1]
        qr = [qrow0, qrow1]

        def zrow(r, _):
            for k in range(D // L):
                prow0[r, pl.ds(k * L, L)] = jnp.zeros((L,), jnp.float32)
            return 0
        lax.fori_loop(0, st, zrow, 0)
        for j in range(n_st):
            pltpu.async_copy(
                prow0, agg_sh.at[pl.ds(sid * rows_w + j * st, st)], semg)
        for j in range(n_st):
            pltpu.make_async_copy(
                prow0, agg_sh.at[pl.ds(sid * rows_w, st)], semg).wait()
        plsc.subcore_barrier()

        base_w = (cid * NS + sid) * ew

        def load_super(s):
            base = base_w + s * (SBC * CH)
            pltpu.sync_copy(ei_hbm.at[pl.ds(base, SBC * CH)], sidx_sc)
            pltpu.sync_copy(ei_hbm.at[pl.ds(E + base, SBC * CH)], didx_sc)

        def fire_gather(c, b):
            pltpu.async_copy(
                p_hbm.at[sidx_sc.at[pl.ds(c * CH, CH)]], pr[b], semg)
            pltpu.async_copy(
                q_hbm.at[didx_sc.at[pl.ds(c * CH, CH)]], qr[b], semg)

        def wait_gather(b):
            pltpu.make_async_copy(
                p_hbm.at[sidx_sc.at[pl.ds(0, CH)]], pr[b], semg).wait()
            pltpu.make_async_copy(
                q_hbm.at[didx_sc.at[pl.ds(0, CH)]], qr[b], semg).wait()

        def fire_scatter(b):
            pltpu.async_copy(pr[b], agg_sh.at[dxb[b]], sems, add=True)

        def wait_scatter(b):
            pltpu.make_async_copy(pr[b], agg_sh.at[dxb[b]], sems).wait()

        def stage_didx(c, b):
            for k in range(CH // L):
                dxb[b][pl.ds(k * L, L)] = didx_sc[pl.ds(c * CH + k * L, L)]

        def compute(b):
            @plsc.parallel_loop(0, CH, unroll=2)
            def _(r):
                for k in range(D // L):
                    s = pl.ds(k * L, L)
                    pr[b][r, s] = jnp.maximum(pr[b][r, s] + qr[b][r, s], 0.0)

        def full_step(c, b, prefetch, pending_scatter):
            if prefetch:
                if pending_scatter:
                    wait_scatter(1 - b)
                fire_gather(c + 1, 1 - b)
            wait_gather(b)
            stage_didx(c, b)
            compute(b)
            fire_scatter(b)

        def super_body(s, first):
            load_super(s)
            if not first:
                wait_scatter(0)
            fire_gather(0, 0)
            full_step(0, 0, prefetch=True, pending_scatter=not first)

            def body(j, _):
                c = 2 * j + 1
                full_step(c, 1, prefetch=True, pending_scatter=True)
                full_step(c + 1, 0, prefetch=True, pending_scatter=True)
                return 0
            lax.fori_loop(0, kk, body, 0)

            full_step(SBC - 2, 1, prefetch=True, pending_scatter=True)
            full_step(SBC - 1, 0, prefetch=False, pending_scatter=False)

        super_body(0, first=True)

        def sbody(s, _):
            super_body(s, first=False)
            return 0
        lax.fori_loop(1, n_sup, sbody, 0)
        wait_scatter(1)
        wait_scatter(0)

        plsc.subcore_barrier()
        for j in range(n_st):
            off = sid * rows_w + j * st
            buf = pr[j % 2]
            if j >= 2:
                pltpu.make_async_copy(
                    buf, out_hbm.at[cid, pl.ds(0, st)], semg).wait()
            pltpu.sync_copy(agg_sh.at[pl.ds(off, st)], buf)
            pltpu.async_copy(buf, out_hbm.at[cid, pl.ds(off, st)], semg)
        pltpu.make_async_copy(
            prow0, out_hbm.at[cid, pl.ds(0, st)], semg).wait()
        pltpu.make_async_copy(
            prow1, out_hbm.at[cid, pl.ds(0, st)], semg).wait()

    return sc_edge


def kernel(x, edge_index, W_msg, b_msg, W_upd, b_upd):
    N, D = x.shape
    E = edge_index.shape[1]

    P, Q = pl.pallas_call(
        _pre_body,
        out_shape=[jax.ShapeDtypeStruct((N, D), jnp.float32)] * 2,
    )(x, W_msg[:D], W_msg[D:], b_msg.reshape(1, D))

    agg = _make_sc_edge(N, D, E)(P, Q, edge_index.reshape(2 * E))

    out = pl.pallas_call(
        _upd_body,
        out_shape=jax.ShapeDtypeStruct((N, D), jnp.float32),
    )(agg, x, W_upd[:D], W_upd[D:], b_upd.reshape(1, D))
    return out
